# Initial kernel scaffold; baseline (speedup 1.0000x reference)
import jax
import jax.numpy as jnp
from jax.experimental import pallas as pl


def kernel(pos_triplets, neg_triplets, e_table, r_table):
    raise NotImplementedError("write your pallas kernel here")


# speedup vs baseline: 1.1915x; 1.1915x over previous
import functools

import jax
import jax.numpy as jnp
from jax import lax
from jax.experimental import pallas as pl
from jax.experimental.pallas import tpu as pltpu
from jax.experimental.pallas import tpu_sc as plsc

DIM = 64
BATCH = 16384
MARGIN = 1.0

NUM_CORES = 2
NUM_SUBCORES = 16
NUM_WORKERS = NUM_CORES * NUM_SUBCORES
ROWS_PER_W = BATCH // NUM_WORKERS
CHUNK = 128
CHUNKS_PER_HALF = ROWS_PER_W // CHUNK
GROUPS = CHUNK // 16


def _sc_body(idx_hbm, e_hbm, r_hbm, loss_hbm, pos_hbm, neg_hbm,
             idx_v, hbuf, rbuf, tbuf, dist_v, loss_v, sem):
    wid = lax.axis_index("s") * NUM_CORES + lax.axis_index("c")
    base = wid * ROWS_PER_W

    pltpu.sync_copy(idx_hbm.at[pl.ds(wid * CHUNKS_PER_HALF, CHUNKS_PER_HALF)],
                    idx_v)

    lane = lax.iota(jnp.int32, 16)

    def chunk_body(c, _):
        cc = c % CHUNKS_PER_HALF
        kind0 = (c // CHUNKS_PER_HALF) * 3

        dh = pltpu.async_copy(e_hbm.at[idx_v.at[cc, kind0]], hbuf, sem)
        dr = pltpu.async_copy(r_hbm.at[idx_v.at[cc, kind0 + 1]], rbuf, sem)
        dt = pltpu.async_copy(e_hbm.at[idx_v.at[cc, kind0 + 2]], tbuf, sem)
        dh.wait()
        dr.wait()
        dt.wait()

        def group_body(g, _):
            rows = g * 16 + lane
            nh = jnp.zeros((16,), jnp.float32)
            nt = jnp.zeros((16,), jnp.float32)
            for d in range(DIM):
                col = (lane + d) & (DIM - 1)
                nh = nh + jnp.abs(plsc.load_gather(hbuf, [rows, col]))
                nt = nt + jnp.abs(plsc.load_gather(tbuf, [rows, col]))
            inv_nh = 1.0 / nh
            inv_nt = 1.0 / nt
            acc = jnp.zeros((16,), jnp.float32)
            for d in range(DIM):
                col = (lane + d) & (DIM - 1)
                hv = plsc.load_gather(hbuf, [rows, col])
                rv = plsc.load_gather(rbuf, [rows, col])
                tv = plsc.load_gather(tbuf, [rows, col])
                acc = acc + jnp.abs(hv * inv_nh + rv - tv * inv_nt)
            dist_v[pl.ds(c * CHUNK + g * 16, 16)] = acc
            return 0

        lax.fori_loop(0, GROUPS, group_body, 0)
        return 0

    lax.fori_loop(0, 2 * CHUNKS_PER_HALF, chunk_body, 0)

    for v in range(ROWS_PER_W // 16):
        pv = dist_v[pl.ds(v * 16, 16)]
        nv = dist_v[pl.ds(ROWS_PER_W + v * 16, 16)]
        loss_v[pl.ds(v * 16, 16)] = jnp.maximum(pv - nv + MARGIN, 0.0)

    pltpu.sync_copy(loss_v, loss_hbm.at[pl.ds(base, ROWS_PER_W)])
    pltpu.sync_copy(dist_v.at[pl.ds(0, ROWS_PER_W)],
                    pos_hbm.at[pl.ds(base, ROWS_PER_W)])
    pltpu.sync_copy(dist_v.at[pl.ds(ROWS_PER_W, ROWS_PER_W)],
                    neg_hbm.at[pl.ds(base, ROWS_PER_W)])


@functools.partial(
    pl.kernel,
    out_type=(
        jax.ShapeDtypeStruct((BATCH,), jnp.float32),
        jax.ShapeDtypeStruct((BATCH,), jnp.float32),
        jax.ShapeDtypeStruct((BATCH,), jnp.float32),
    ),
    mesh=plsc.VectorSubcoreMesh(core_axis_name="c", subcore_axis_name="s",
                                num_cores=NUM_CORES,
                                num_subcores=NUM_SUBCORES),
    scratch_types=[
        pltpu.VMEM((CHUNKS_PER_HALF, 6, CHUNK), jnp.int32),
        pltpu.VMEM((CHUNK, DIM), jnp.float32),
        pltpu.VMEM((CHUNK, DIM), jnp.float32),
        pltpu.VMEM((CHUNK, DIM), jnp.float32),
        pltpu.VMEM((2 * ROWS_PER_W,), jnp.float32),
        pltpu.VMEM((ROWS_PER_W,), jnp.float32),
        pltpu.SemaphoreType.DMA,
    ],
    compiler_params=pltpu.CompilerParams(needs_layout_passes=False,
                                         use_tc_tiling_on_sc=False),
)
def _transe_sc(idx_hbm, e_hbm, r_hbm, loss_hbm, pos_hbm, neg_hbm,
               idx_v, hbuf, rbuf, tbuf, dist_v, loss_v, sem):
    _sc_body(idx_hbm, e_hbm, r_hbm, loss_hbm, pos_hbm, neg_hbm,
             idx_v, hbuf, rbuf, tbuf, dist_v, loss_v, sem)


def kernel(pos_triplets, neg_triplets, e_table, r_table):
    p = pos_triplets.astype(jnp.int32).reshape(BATCH // CHUNK, CHUNK, 3)
    n = neg_triplets.astype(jnp.int32).reshape(BATCH // CHUNK, CHUNK, 3)
    idx = jnp.concatenate([p.transpose(0, 2, 1), n.transpose(0, 2, 1)],
                          axis=1)
    loss, pos_d, neg_d = _transe_sc(idx, e_table, r_table)
    return (loss, pos_d, neg_d)
